# Initial kernel scaffold; baseline (speedup 1.0000x reference)
import jax
import jax.numpy as jnp
from jax.experimental import pallas as pl


def kernel(x, edge_index, edge_attr, Wq, bq, Wk, bk, Wv, bv, We, Wskip, bskip, Wbeta, Wt, bt, gamma, beta_bn):
    raise NotImplementedError("write your pallas kernel here")


# speedup vs baseline: 3.8642x; 3.8642x over previous
import functools

import jax
import jax.numpy as jnp
import numpy as np
from jax import lax
from jax.experimental import pallas as pl
from jax.experimental.pallas import tpu as pltpu
from jax.experimental.pallas import tpu_sc as plsc

_N, _E, _F, _H, _C, _D = 10000, 320000, 128, 4, 128, 512
_NC, _NS, _NW = 2, 16, 32
_EPW = _E // _NW
_EPS = _E // _NS
_B1 = 80
_NB1 = _EPW // _B1
_B2 = 256
_NB2 = _EPS // _B2
_R2 = _EPS - _NB2 * _B2
_RN = _N // 4
_DEN_PAD = 40960
_CV_PAD = 20480
_SLAB_ROWS = 2560
_ISC = float(1.0 / np.sqrt(_C))


def _sc_mesh():
    return plsc.VectorSubcoreMesh(
        core_axis_name="c", subcore_axis_name="s",
        num_cores=_NC, num_subcores=_NS)


def _tc_proj(x, wcat_t, bcat):
    bn, g = 400, 25

    def body(x_ref, w_ref, b_ref, q_ref, k_ref, v_ref, xr_ref):
        y = jnp.dot(x_ref[...], w_ref[...],
                    preferred_element_type=jnp.float32) + b_ref[...]
        q_ref[...] = y[:, 0 * _D:1 * _D]
        k_ref[...] = y[:, 1 * _D:2 * _D]
        v_ref[...] = y[:, 2 * _D:3 * _D]
        xr_ref[...] = y[:, 3 * _D:4 * _D]

    return pl.pallas_call(
        body,
        grid=(g,),
        in_specs=[
            pl.BlockSpec((bn, _F), lambda i: (i, 0)),
            pl.BlockSpec((_F, 4 * _D), lambda i: (0, 0)),
            pl.BlockSpec((1, 4 * _D), lambda i: (0, 0)),
        ],
        out_specs=[pl.BlockSpec((bn, _D), lambda i: (i, 0))] * 4,
        out_shape=[jax.ShapeDtypeStruct((_N, _D), jnp.float32)] * 4,
    )(x, wcat_t, bcat.reshape(1, 4 * _D))


def _sc_pass1(q, k, src, dst, attr, we_vec, z1d):
    @functools.partial(
        pl.kernel,
        out_type=[jax.ShapeDtypeStruct((_E * _H,), jnp.float32),
                  jax.ShapeDtypeStruct((_NC, _DEN_PAD), jnp.float32)],
        mesh=_sc_mesh(),
        compiler_params=pltpu.CompilerParams(use_tc_tiling_on_sc=False, needs_layout_passes=False),
        scratch_types=[
            pltpu.VMEM((_B1, _D), jnp.float32),
            pltpu.VMEM((_B1, _D), jnp.float32),
            pltpu.VMEM((_B1,), jnp.int32),
            pltpu.VMEM((_B1,), jnp.int32),
            pltpu.VMEM((_B1,), jnp.float32),
            pltpu.VMEM((_B1 * _H,), jnp.float32),
            pltpu.VMEM((_B1 * _H,), jnp.int32),
            pltpu.VMEM((_D,), jnp.float32),
            pltpu.VMEM_SHARED((_DEN_PAD,), jnp.float32),
            pltpu.SemaphoreType.DMA,
            pltpu.SemaphoreType.DMA,
        ])
    def k2(q_h, k_h, src_h, dst_h, attr_h, we_h, z_h, ex_o, den_o,
           qrows, krows, srcb, dstb, attrb, exstage, denidx, wev, den_sh,
           sem0, sem1):
        c = lax.axis_index("c")
        s = lax.axis_index("s")
        w = c * _NS + s
        pltpu.sync_copy(we_h, wev)
        chunk = _DEN_PAD // _NS
        pltpu.sync_copy(z_h.at[pl.ds(s * chunk, chunk)],
                        den_sh.at[pl.ds(s * chunk, chunk)])
        plsc.subcore_barrier()

        def block(i, carry):
            base = w * _EPW + i * _B1
            pltpu.sync_copy(src_h.at[pl.ds(base, _B1)], srcb)
            pltpu.sync_copy(dst_h.at[pl.ds(base, _B1)], dstb)
            pltpu.sync_copy(attr_h.at[pl.ds(base, _B1)], attrb)
            cp0 = pltpu.async_copy(q_h.at[dstb], qrows, sem0)
            cp1 = pltpu.async_copy(k_h.at[srcb], krows, sem1)
            cp0.wait()
            cp1.wait()
            for gix in range(_B1 // 16):
                rows = lax.iota(jnp.int32, 16) + gix * 16
                attrv = attrb[pl.ds(gix * 16, 16)]
                dstv = dstb[pl.ds(gix * 16, 16)]
                for h in range(_H):
                    def cbody(c16, acc, h=h, rows=rows, attrv=attrv):
                        col0 = h * _C + c16 * 16
                        wch = wev[pl.ds(col0, 16)]
                        for cc in range(16):
                            colv = jnp.full((16,), col0 + cc, jnp.int32)
                            qv = plsc.load_gather(qrows, [rows, colv])
                            kv = plsc.load_gather(krows, [rows, colv])
                            acc = acc + qv * (kv + attrv * wch[cc])
                        return acc
                    acc = lax.fori_loop(0, _C // 16, cbody,
                                        jnp.zeros((16,), jnp.float32))
                    ex = jnp.exp(acc * _ISC)
                    plsc.store_scatter(exstage, [rows * _H + h], ex)
                    plsc.store_scatter(denidx, [rows * _H + h],
                                       dstv * _H + h)
            pltpu.sync_copy(exstage, ex_o.at[pl.ds(base * _H, _B1 * _H)])
            pltpu.async_copy(exstage, den_sh.at[denidx], sem0,
                             add=True).wait()
            return carry

        lax.fori_loop(0, _NB1, block, 0)
        plsc.subcore_barrier()

        @pl.when(s == 0)
        def _():
            pltpu.sync_copy(den_sh, den_o.at[c])

    return k2(q, k, src, dst, attr, we_vec, z1d)


def _sc_pass2(v, src, dst, attr, exb, den, z1d, z2d):
    @functools.partial(
        pl.kernel,
        out_type=[jax.ShapeDtypeStruct((_N, _D), jnp.float32),
                  jax.ShapeDtypeStruct((_NC, _CV_PAD), jnp.float32)],
        mesh=_sc_mesh(),
        compiler_params=pltpu.CompilerParams(use_tc_tiling_on_sc=False, needs_layout_passes=False),
        scratch_types=[
            pltpu.VMEM((_B2,), jnp.int32),
            pltpu.VMEM((_B2,), jnp.int32),
            pltpu.VMEM((_B2,), jnp.float32),
            pltpu.VMEM((_B2 * _H,), jnp.float32),
            pltpu.VMEM((_B2 + 16,), jnp.int32),
            pltpu.VMEM((_B2 + 16,), jnp.int32),
            pltpu.VMEM((_B2 + 16,), jnp.float32),
            pltpu.VMEM((_H * (_B2 + 16),), jnp.float32),
            pltpu.VMEM((16, _D), jnp.float32),
            pltpu.VMEM((16 * _H,), jnp.float32),
            pltpu.VMEM((16 * _H,), jnp.float32),
            pltpu.VMEM((16 * _H,), jnp.int32),
            pltpu.VMEM((_RN * _H,), jnp.float32),
            pltpu.VMEM((_RN * _H,), jnp.float32),
            pltpu.VMEM_SHARED((_SLAB_ROWS, _D), jnp.float32),
            pltpu.VMEM_SHARED((_CV_PAD,), jnp.float32),
            pltpu.SemaphoreType.DMA,
            pltpu.SemaphoreType.DMA,
            pltpu.SemaphoreType.DMA,
        ])
    def k3(v_h, src_h, dst_h, attr_h, ex_h, den_h, z1_h, z2_h, out_o, cv_o,
           dstb, srcb, attrb, exbuf, cdst, csrc, cattr, cexs, vbuf,
           attnst, cvst, cvidx, invd, tmpd, slab_sh, cv_sh,
           semv, semc, sems):
        c = lax.axis_index("c")
        s = lax.axis_index("s")
        lanes = lax.iota(jnp.int32, 16)
        zf16 = jnp.zeros((16,), jnp.float32)
        zi16 = jnp.zeros((16,), jnp.int32)

        cvchunk = _CV_PAD // _NS
        pltpu.sync_copy(z1_h.at[pl.ds(s * cvchunk, cvchunk)],
                        cv_sh.at[pl.ds(s * cvchunk, cvchunk)])

        def scan_block(base, nedge, glo, rr):
            pltpu.sync_copy(src_h.at[pl.ds(base, nedge)],
                            srcb.at[pl.ds(0, nedge)])
            pltpu.sync_copy(dst_h.at[pl.ds(base, nedge)],
                            dstb.at[pl.ds(0, nedge)])
            pltpu.sync_copy(attr_h.at[pl.ds(base, nedge)],
                            attrb.at[pl.ds(0, nedge)])
            pltpu.sync_copy(ex_h.at[pl.ds(base * _H, nedge * _H)],
                            exbuf.at[pl.ds(0, nedge * _H)])
            cur = jnp.int32(0)
            for gix in range(nedge // 16):
                dstv = dstb[pl.ds(gix * 16, 16)]
                srcv = srcb[pl.ds(gix * 16, 16)]
                attrv = attrb[pl.ds(gix * 16, 16)]
                m = (dstv >= glo) & (dstv < glo + _RN)
                plsc.store_compressed(cdst.at[pl.ds(cur, 16)], dstv - glo,
                                      mask=m)
                plsc.store_compressed(csrc.at[pl.ds(cur, 16)], srcv, mask=m)
                plsc.store_compressed(cattr.at[pl.ds(cur, 16)], attrv,
                                      mask=m)
                eix = (lanes + gix * 16) * _H
                for h in range(_H):
                    exv = plsc.load_gather(exbuf, [eix + h])
                    plsc.store_compressed(
                        cexs.at[pl.ds(h * (_B2 + 16) + cur, 16)], exv,
                        mask=m)
                cur = cur + plsc.all_reduce_population_count(m)[0]

            cdst[pl.ds(cur, 16)] = zi16
            csrc[pl.ds(cur, 16)] = zi16
            cattr[pl.ds(cur, 16)] = zf16
            for h in range(_H):
                cexs[pl.ds(h * (_B2 + 16) + cur, 16)] = zf16

            def sub(b, carry, rr=rr):
                off = b * 16
                dlv = cdst[pl.ds(off, 16)]
                srcv = csrc[pl.ds(off, 16)]
                attrv = cattr[pl.ds(off, 16)]
                cp = pltpu.async_copy(v_h.at[srcv], vbuf, semv)
                for h in range(_H):
                    iv = plsc.load_gather(invd, [dlv * _H + h])
                    av = cexs[pl.ds(h * (_B2 + 16) + off, 16)] * iv
                    plsc.store_scatter(attnst, [lanes * _H + h], av)
                    plsc.store_scatter(cvst, [lanes * _H + h], av * attrv)
                    plsc.store_scatter(cvidx, [lanes * _H + h],
                                       rr * (_CV_PAD // 2) + dlv * _H + h)
                pltpu.async_copy(cvst, cv_sh.at[cvidx], semc,
                                 add=True).wait()
                cp.wait()
                for e in range(16):
                    avec = attnst[pl.ds((e // 4) * 16, 16)] if e % 4 == 0 \
                        else avec
                    for h in range(_H):
                        a_s = avec[(e % 4) * _H + h]
                        for c16 in range(_C // 16):
                            col0 = h * _C + c16 * 16
                            vbuf[e, pl.ds(col0, 16)] = (
                                vbuf[e, pl.ds(col0, 16)] * a_s)
                pltpu.async_copy(vbuf, slab_sh.at[dlv], sems,
                                 add=True).wait()
                return carry

            nb = (cur + 15) // 16
            lax.fori_loop(0, nb, sub, 0)

        for rr in range(2):
            glo = (c * 2 + rr) * _RN
            rows = _SLAB_ROWS // _NS
            pltpu.sync_copy(z2_h.at[pl.ds(s * rows, rows)],
                            slab_sh.at[pl.ds(s * rows, rows)])
            pltpu.sync_copy(den_h.at[0, pl.ds(glo * _H, _RN * _H)], invd)
            pltpu.sync_copy(den_h.at[1, pl.ds(glo * _H, _RN * _H)], tmpd)

            def dinv(i, carry):
                dsum = invd[pl.ds(i * 16, 16)] + tmpd[pl.ds(i * 16, 16)]
                invd[pl.ds(i * 16, 16)] = 1.0 / (dsum + 1e-16)
                return carry

            lax.fori_loop(0, _RN * _H // 16, dinv, 0)
            plsc.subcore_barrier()

            def outer(i, carry, glo=glo, rr=rr):
                scan_block(s * _EPS + i * _B2, _B2, glo, rr)
                return carry

            lax.fori_loop(0, _NB2, outer, 0)
            if _R2:
                scan_block(s * _EPS + _NB2 * _B2, _R2, glo, rr)
            plsc.subcore_barrier()

            @pl.when(s == 0)
            def _(glo=glo):
                pltpu.sync_copy(slab_sh.at[pl.ds(0, _RN)],
                                out_o.at[pl.ds(glo, _RN)])
            plsc.subcore_barrier()

        @pl.when(s == 0)
        def _():
            pltpu.sync_copy(cv_sh, cv_o.at[c])

    return k3(v, src, dst, attr, exb, den, z1d, z2d)


def _tc_finish(out_msg, cvn, xr, we_bd, w_out, w_xr, wt_t, bt, gamma, beta_bn):
    bn, g = 400, 25

    def body(om_ref, cv_ref, xr_ref, webd_ref, wo_ref, wxr_ref, wt_ref,
             bt_ref, g_ref, bb_ref, o_ref, hall, sums):
        p = pl.program_id(0)
        i = pl.program_id(1)

        @pl.when(p == 0)
        def _():
            om = om_ref[...] + jnp.dot(cv_ref[...], webd_ref[...],
                                       preferred_element_type=jnp.float32)
            xrb = xr_ref[...]
            sb = (jnp.dot(om, wo_ref[...],
                          preferred_element_type=jnp.float32)
                  + jnp.dot(xrb, wxr_ref[...],
                            preferred_element_type=jnp.float32))
            bg = jax.nn.sigmoid(sb)
            out2 = bg * xrb + (1.0 - bg) * om
            hb = jnp.dot(out2, wt_ref[...],
                         preferred_element_type=jnp.float32) + bt_ref[...]

            @pl.when(i == 0)
            def _():
                sums[...] = jnp.zeros_like(sums)

            sums[0:1, :] += jnp.sum(hb, axis=0, keepdims=True)
            sums[1:2, :] += jnp.sum(hb * hb, axis=0, keepdims=True)
            hall[pl.ds(i * bn, bn), :] = hb

        @pl.when(p == 1)
        def _():
            mu = sums[0:1, :] * (1.0 / _N)
            var = sums[1:2, :] * (1.0 / _N) - mu * mu
            rstd = lax.rsqrt(var + 1e-5)
            o_ref[...] = ((hall[pl.ds(i * bn, bn), :] - mu) * rstd
                          * g_ref[...] + bb_ref[...])

    full = lambda shape: pl.BlockSpec(shape, lambda p, i: tuple(0 for _ in shape))
    return pl.pallas_call(
        body,
        grid=(2, g),
        in_specs=[
            pl.BlockSpec((bn, _D), lambda p, i: (i, 0)),
            pl.BlockSpec((bn, _H), lambda p, i: (i, 0)),
            pl.BlockSpec((bn, _D), lambda p, i: (i, 0)),
            full((_H, _D)),
            full((_D, 1)),
            full((_D, 1)),
            full((_D, _F)),
            full((1, _F)),
            full((1, _F)),
            full((1, _F)),
        ],
        out_specs=pl.BlockSpec((bn, _F), lambda p, i: (i, 0)),
        out_shape=jax.ShapeDtypeStruct((_N, _F), jnp.float32),
        scratch_shapes=[pltpu.VMEM((_N, _F), jnp.float32),
                        pltpu.VMEM((2, _F), jnp.float32)],
    )(out_msg, cvn, xr, we_bd, w_out, w_xr, wt_t, bt, gamma, beta_bn)


def kernel(x, edge_index, edge_attr, Wq, bq, Wk, bk, Wv, bv, We, Wskip,
           bskip, Wbeta, Wt, bt, gamma, beta_bn):
    src = edge_index[0]
    dst = edge_index[1]
    attr = edge_attr[:, 0]
    wcat_t = jnp.concatenate([Wq, Wk, Wv, Wskip], axis=0).T
    bcat = jnp.concatenate([bq, bk, bv, bskip])
    q, k, v, xr = _tc_proj(x, wcat_t, bcat)

    we_vec = We[:, 0]
    z1d = jnp.zeros((_DEN_PAD,), jnp.float32)
    z2d = jnp.zeros((_SLAB_ROWS, _D), jnp.float32)
    exb, den = _sc_pass1(q, k, src, dst, attr, we_vec, z1d)
    out_msg, cv = _sc_pass2(v, src, dst, attr, exb, den, z1d, z2d)

    cvn = cv.reshape(_NC, 2, _CV_PAD // 2)[:, :, :_RN * _H]
    cvn = cvn.reshape(_N, _H)
    we_bd = jnp.zeros((_H, _D), jnp.float32)
    for h in range(_H):
        we_bd = we_bd.at[h, h * _C:(h + 1) * _C].set(
            we_vec[h * _C:(h + 1) * _C])
    w_out = (Wbeta[0, :_D] + Wbeta[0, 2 * _D:]).reshape(_D, 1)
    w_xr = (Wbeta[0, _D:2 * _D] - Wbeta[0, 2 * _D:]).reshape(_D, 1)
    return _tc_finish(out_msg, cvn, xr, we_bd, w_out, w_xr, Wt.T,
                      bt.reshape(1, _F), gamma.reshape(1, _F),
                      beta_bn.reshape(1, _F))


# speedup vs baseline: 6.7564x; 1.7485x over previous
import functools

import jax
import jax.numpy as jnp
import numpy as np
from jax import lax
from jax.experimental import pallas as pl
from jax.experimental.pallas import tpu as pltpu
from jax.experimental.pallas import tpu_sc as plsc

_N, _E, _F, _H, _C, _D = 10000, 320000, 128, 4, 128, 512
_NC, _NS, _NW = 2, 16, 32
_EPW = _E // _NW
_EPS = _E // _NS
_B1 = 80
_NB1 = _EPW // _B1
_B2 = 256
_NB2 = _EPS // _B2
_R2 = _EPS - _NB2 * _B2
_RN = _N // 4
_DEN_PAD = 40960
_CV_PAD = 20480
_SLAB_ROWS = 2560
_ISC = float(1.0 / np.sqrt(_C))


def _sc_mesh():
    return plsc.VectorSubcoreMesh(
        core_axis_name="c", subcore_axis_name="s",
        num_cores=_NC, num_subcores=_NS)


def _tc_proj(x, wcat_t, bcat, we_mat):
    bn, g = 400, 25

    def body(x_ref, w_ref, b_ref, wem_ref, q_ref, k_ref, v_ref, xr_ref,
             g_ref):
        y = jnp.dot(x_ref[...], w_ref[...],
                    preferred_element_type=jnp.float32) + b_ref[...]
        q_ref[...] = y[:, 0 * _D:1 * _D]
        k_ref[...] = y[:, 1 * _D:2 * _D]
        v_ref[...] = y[:, 2 * _D:3 * _D]
        xr_ref[...] = y[:, 3 * _D:4 * _D]
        g_ref[...] = jnp.dot(y[:, 0 * _D:1 * _D], wem_ref[...],
                             preferred_element_type=jnp.float32)

    return pl.pallas_call(
        body,
        grid=(g,),
        in_specs=[
            pl.BlockSpec((bn, _F), lambda i: (i, 0)),
            pl.BlockSpec((_F, 4 * _D), lambda i: (0, 0)),
            pl.BlockSpec((1, 4 * _D), lambda i: (0, 0)),
            pl.BlockSpec((_D, 16), lambda i: (0, 0)),
        ],
        out_specs=[pl.BlockSpec((bn, _D), lambda i: (i, 0))] * 4
        + [pl.BlockSpec((bn, 16), lambda i: (i, 0))],
        out_shape=[jax.ShapeDtypeStruct((_N, _D), jnp.float32)] * 4
        + [jax.ShapeDtypeStruct((_N, 16), jnp.float32)],
    )(x, wcat_t, bcat.reshape(1, 4 * _D), we_mat)


def _sc_pass1(q, k, src, dst, attr, g2, z1d):
    @functools.partial(
        pl.kernel,
        out_type=[jax.ShapeDtypeStruct((_E * _H,), jnp.float32),
                  jax.ShapeDtypeStruct((_NC, _DEN_PAD), jnp.float32)],
        mesh=_sc_mesh(),
        compiler_params=pltpu.CompilerParams(use_tc_tiling_on_sc=False, needs_layout_passes=False),
        scratch_types=[
            pltpu.VMEM((_B1, _D), jnp.float32),
            pltpu.VMEM((_B1, _D), jnp.float32),
            pltpu.VMEM((_B1,), jnp.int32),
            pltpu.VMEM((_B1,), jnp.int32),
            pltpu.VMEM((_B1,), jnp.float32),
            pltpu.VMEM((_B1, 16), jnp.float32),
            pltpu.VMEM((16 * _H * 16,), jnp.float32),
            pltpu.VMEM((_B1 * _H,), jnp.float32),
            pltpu.VMEM((_B1 * _H,), jnp.int32),
            pltpu.VMEM_SHARED((_DEN_PAD,), jnp.float32),
            pltpu.SemaphoreType.DMA,
            pltpu.SemaphoreType.DMA,
            pltpu.SemaphoreType.DMA,
        ])
    def k2(q_h, k_h, src_h, dst_h, attr_h, g_h, z_h, ex_o, den_o,
           qrows, krows, srcb, dstb, attrb, gblk, pstage, exstage, denidx,
           den_sh, sem0, sem1, sem2):
        c = lax.axis_index("c")
        s = lax.axis_index("s")
        w = c * _NS + s
        chunk = _DEN_PAD // _NS
        pltpu.sync_copy(z_h.at[pl.ds(s * chunk, chunk)],
                        den_sh.at[pl.ds(s * chunk, chunk)])
        plsc.subcore_barrier()
        lanes = lax.iota(jnp.int32, 16)
        lanes64 = lanes * (16 * _H)

        def block(i, carry):
            base = w * _EPW + i * _B1
            pltpu.sync_copy(src_h.at[pl.ds(base, _B1)], srcb)
            pltpu.sync_copy(dst_h.at[pl.ds(base, _B1)], dstb)
            pltpu.sync_copy(attr_h.at[pl.ds(base, _B1)], attrb)
            cp0 = pltpu.async_copy(q_h.at[dstb], qrows, sem0)
            cp1 = pltpu.async_copy(k_h.at[srcb], krows, sem1)
            cpg = pltpu.async_copy(g_h.at[dstb], gblk, sem2)
            cp0.wait()
            cp1.wait()
            cpg.wait()

            def group(gix, carry2):
                g0 = gix * 16
                for e16 in range(16):
                    e = g0 + e16
                    for h in range(_H):
                        col0 = h * _C
                        acc = (qrows[e, pl.ds(col0, 16)]
                               * krows[e, pl.ds(col0, 16)])
                        for c16 in range(1, _C // 16):
                            cs = col0 + c16 * 16
                            acc = acc + (qrows[e, pl.ds(cs, 16)]
                                         * krows[e, pl.ds(cs, 16)])
                        pstage[pl.ds((e16 * _H + h) * 16, 16)] = acc
                rows = lanes + g0
                rows4 = rows * _H
                attrv = attrb[pl.ds(g0, 16)]
                dstv = dstb[pl.ds(g0, 16)]
                dstv4 = dstv * _H
                for h in range(_H):
                    ivec = lanes64 + h * 16
                    dots = plsc.load_gather(pstage, [ivec])
                    for j in range(1, 16):
                        dots = dots + plsc.load_gather(pstage, [ivec + j])
                    gv = plsc.load_gather(
                        gblk, [rows, jnp.full((16,), h, jnp.int32)])
                    ex = jnp.exp((dots + attrv * gv) * _ISC)
                    plsc.store_scatter(exstage, [rows4 + h], ex)
                    plsc.store_scatter(denidx, [rows4 + h], dstv4 + h)
                return carry2

            lax.fori_loop(0, _B1 // 16, group, 0)
            pltpu.sync_copy(exstage, ex_o.at[pl.ds(base * _H, _B1 * _H)])
            pltpu.async_copy(exstage, den_sh.at[denidx], sem0,
                             add=True).wait()
            return carry

        lax.fori_loop(0, _NB1, block, 0)
        plsc.subcore_barrier()

        @pl.when(s == 0)
        def _():
            pltpu.sync_copy(den_sh, den_o.at[c])

    return k2(q, k, src, dst, attr, g2, z1d)


def _sc_pass2(v, src, dst, attr, exb, den, z1d, z2d):
    @functools.partial(
        pl.kernel,
        out_type=[jax.ShapeDtypeStruct((_N, _D), jnp.float32),
                  jax.ShapeDtypeStruct((_NC, _CV_PAD), jnp.float32)],
        mesh=_sc_mesh(),
        compiler_params=pltpu.CompilerParams(use_tc_tiling_on_sc=False, needs_layout_passes=False),
        scratch_types=[
            pltpu.VMEM((_B2,), jnp.int32),
            pltpu.VMEM((_B2,), jnp.int32),
            pltpu.VMEM((_B2,), jnp.float32),
            pltpu.VMEM((_B2 * _H,), jnp.float32),
            pltpu.VMEM((_B2 + 16,), jnp.int32),
            pltpu.VMEM((_B2 + 16,), jnp.int32),
            pltpu.VMEM((_B2 + 16,), jnp.float32),
            pltpu.VMEM((_H * (_B2 + 16),), jnp.float32),
            pltpu.VMEM((16, _D), jnp.float32),
            pltpu.VMEM((16 * _H,), jnp.float32),
            pltpu.VMEM((16 * _H,), jnp.float32),
            pltpu.VMEM((16 * _H,), jnp.int32),
            pltpu.VMEM((_RN * _H,), jnp.float32),
            pltpu.VMEM((_RN * _H,), jnp.float32),
            pltpu.VMEM_SHARED((_SLAB_ROWS, _D), jnp.float32),
            pltpu.VMEM_SHARED((_CV_PAD,), jnp.float32),
            pltpu.SemaphoreType.DMA,
            pltpu.SemaphoreType.DMA,
            pltpu.SemaphoreType.DMA,
        ])
    def k3(v_h, src_h, dst_h, attr_h, ex_h, den_h, z1_h, z2_h, out_o, cv_o,
           dstb, srcb, attrb, exbuf, cdst, csrc, cattr, cexs, vbuf,
           attnst, cvst, cvidx, invd, tmpd, slab_sh, cv_sh,
           semv, semc, sems):
        c = lax.axis_index("c")
        s = lax.axis_index("s")
        lanes = lax.iota(jnp.int32, 16)
        zf16 = jnp.zeros((16,), jnp.float32)
        zi16 = jnp.zeros((16,), jnp.int32)

        cvchunk = _CV_PAD // _NS
        pltpu.sync_copy(z1_h.at[pl.ds(s * cvchunk, cvchunk)],
                        cv_sh.at[pl.ds(s * cvchunk, cvchunk)])

        def scan_block(base, nedge, glo, rr):
            pltpu.sync_copy(src_h.at[pl.ds(base, nedge)],
                            srcb.at[pl.ds(0, nedge)])
            pltpu.sync_copy(dst_h.at[pl.ds(base, nedge)],
                            dstb.at[pl.ds(0, nedge)])
            pltpu.sync_copy(attr_h.at[pl.ds(base, nedge)],
                            attrb.at[pl.ds(0, nedge)])
            pltpu.sync_copy(ex_h.at[pl.ds(base * _H, nedge * _H)],
                            exbuf.at[pl.ds(0, nedge * _H)])
            cur = jnp.int32(0)
            for gix in range(nedge // 16):
                dstv = dstb[pl.ds(gix * 16, 16)]
                srcv = srcb[pl.ds(gix * 16, 16)]
                attrv = attrb[pl.ds(gix * 16, 16)]
                m = (dstv >= glo) & (dstv < glo + _RN)
                plsc.store_compressed(cdst.at[pl.ds(cur, 16)], dstv - glo,
                                      mask=m)
                plsc.store_compressed(csrc.at[pl.ds(cur, 16)], srcv, mask=m)
                plsc.store_compressed(cattr.at[pl.ds(cur, 16)], attrv,
                                      mask=m)
                eix = (lanes + gix * 16) * _H
                for h in range(_H):
                    exv = plsc.load_gather(exbuf, [eix + h])
                    plsc.store_compressed(
                        cexs.at[pl.ds(h * (_B2 + 16) + cur, 16)], exv,
                        mask=m)
                cur = cur + plsc.all_reduce_population_count(m)[0]

            cdst[pl.ds(cur, 16)] = zi16
            csrc[pl.ds(cur, 16)] = zi16
            cattr[pl.ds(cur, 16)] = zf16
            for h in range(_H):
                cexs[pl.ds(h * (_B2 + 16) + cur, 16)] = zf16

            def sub(b, carry, rr=rr):
                off = b * 16
                dlv = cdst[pl.ds(off, 16)]
                srcv = csrc[pl.ds(off, 16)]
                attrv = cattr[pl.ds(off, 16)]
                cp = pltpu.async_copy(v_h.at[srcv], vbuf, semv)
                for h in range(_H):
                    iv = plsc.load_gather(invd, [dlv * _H + h])
                    av = cexs[pl.ds(h * (_B2 + 16) + off, 16)] * iv
                    plsc.store_scatter(attnst, [lanes * _H + h], av)
                    plsc.store_scatter(cvst, [lanes * _H + h], av * attrv)
                    plsc.store_scatter(cvidx, [lanes * _H + h],
                                       rr * (_CV_PAD // 2) + dlv * _H + h)
                pltpu.async_copy(cvst, cv_sh.at[cvidx], semc,
                                 add=True).wait()
                cp.wait()
                for e in range(16):
                    avec = attnst[pl.ds((e // 4) * 16, 16)] if e % 4 == 0 \
                        else avec
                    for h in range(_H):
                        a_s = avec[(e % 4) * _H + h]
                        for c16 in range(_C // 16):
                            col0 = h * _C + c16 * 16
                            vbuf[e, pl.ds(col0, 16)] = (
                                vbuf[e, pl.ds(col0, 16)] * a_s)
                pltpu.async_copy(vbuf, slab_sh.at[dlv], sems,
                                 add=True).wait()
                return carry

            nb = (cur + 15) // 16
            lax.fori_loop(0, nb, sub, 0)

        for rr in range(2):
            glo = (c * 2 + rr) * _RN
            rows = _SLAB_ROWS // _NS
            pltpu.sync_copy(z2_h.at[pl.ds(s * rows, rows)],
                            slab_sh.at[pl.ds(s * rows, rows)])
            pltpu.sync_copy(den_h.at[0, pl.ds(glo * _H, _RN * _H)], invd)
            pltpu.sync_copy(den_h.at[1, pl.ds(glo * _H, _RN * _H)], tmpd)

            def dinv(i, carry):
                dsum = invd[pl.ds(i * 16, 16)] + tmpd[pl.ds(i * 16, 16)]
                invd[pl.ds(i * 16, 16)] = 1.0 / (dsum + 1e-16)
                return carry

            lax.fori_loop(0, _RN * _H // 16, dinv, 0)
            plsc.subcore_barrier()

            def outer(i, carry, glo=glo, rr=rr):
                scan_block(s * _EPS + i * _B2, _B2, glo, rr)
                return carry

            lax.fori_loop(0, _NB2, outer, 0)
            if _R2:
                scan_block(s * _EPS + _NB2 * _B2, _R2, glo, rr)
            plsc.subcore_barrier()

            @pl.when(s == 0)
            def _(glo=glo):
                pltpu.sync_copy(slab_sh.at[pl.ds(0, _RN)],
                                out_o.at[pl.ds(glo, _RN)])
            plsc.subcore_barrier()

        @pl.when(s == 0)
        def _():
            pltpu.sync_copy(cv_sh, cv_o.at[c])

    return k3(v, src, dst, attr, exb, den, z1d, z2d)


def _tc_finish(out_msg, cvn, xr, we_bd, w_out, w_xr, wt_t, bt, gamma, beta_bn):
    bn, g = 400, 25

    def body(om_ref, cv_ref, xr_ref, webd_ref, wo_ref, wxr_ref, wt_ref,
             bt_ref, g_ref, bb_ref, o_ref, hall, sums):
        p = pl.program_id(0)
        i = pl.program_id(1)

        @pl.when(p == 0)
        def _():
            om = om_ref[...] + jnp.dot(cv_ref[...], webd_ref[...],
                                       preferred_element_type=jnp.float32)
            xrb = xr_ref[...]
            sb = (jnp.dot(om, wo_ref[...],
                          preferred_element_type=jnp.float32)
                  + jnp.dot(xrb, wxr_ref[...],
                            preferred_element_type=jnp.float32))
            bg = jax.nn.sigmoid(sb)
            out2 = bg * xrb + (1.0 - bg) * om
            hb = jnp.dot(out2, wt_ref[...],
                         preferred_element_type=jnp.float32) + bt_ref[...]

            @pl.when(i == 0)
            def _():
                sums[...] = jnp.zeros_like(sums)

            sums[0:1, :] += jnp.sum(hb, axis=0, keepdims=True)
            sums[1:2, :] += jnp.sum(hb * hb, axis=0, keepdims=True)
            hall[pl.ds(i * bn, bn), :] = hb

        @pl.when(p == 1)
        def _():
            mu = sums[0:1, :] * (1.0 / _N)
            var = sums[1:2, :] * (1.0 / _N) - mu * mu
            rstd = lax.rsqrt(var + 1e-5)
            o_ref[...] = ((hall[pl.ds(i * bn, bn), :] - mu) * rstd
                          * g_ref[...] + bb_ref[...])

    full = lambda shape: pl.BlockSpec(shape, lambda p, i: tuple(0 for _ in shape))
    return pl.pallas_call(
        body,
        grid=(2, g),
        in_specs=[
            pl.BlockSpec((bn, _D), lambda p, i: (i, 0)),
            pl.BlockSpec((bn, _H), lambda p, i: (i, 0)),
            pl.BlockSpec((bn, _D), lambda p, i: (i, 0)),
            full((_H, _D)),
            full((_D, 1)),
            full((_D, 1)),
            full((_D, _F)),
            full((1, _F)),
            full((1, _F)),
            full((1, _F)),
        ],
        out_specs=pl.BlockSpec((bn, _F), lambda p, i: (i, 0)),
        out_shape=jax.ShapeDtypeStruct((_N, _F), jnp.float32),
        scratch_shapes=[pltpu.VMEM((_N, _F), jnp.float32),
                        pltpu.VMEM((2, _F), jnp.float32)],
    )(out_msg, cvn, xr, we_bd, w_out, w_xr, wt_t, bt, gamma, beta_bn)


def kernel(x, edge_index, edge_attr, Wq, bq, Wk, bk, Wv, bv, We, Wskip,
           bskip, Wbeta, Wt, bt, gamma, beta_bn):
    src = edge_index[0]
    dst = edge_index[1]
    attr = edge_attr[:, 0]
    wcat_t = jnp.concatenate([Wq, Wk, Wv, Wskip], axis=0).T
    bcat = jnp.concatenate([bq, bk, bv, bskip])
    we_vec = We[:, 0]
    we_bd = jnp.zeros((_H, _D), jnp.float32)
    for h in range(_H):
        we_bd = we_bd.at[h, h * _C:(h + 1) * _C].set(
            we_vec[h * _C:(h + 1) * _C])
    wem16 = jnp.zeros((_D, 16), jnp.float32).at[:, :_H].set(we_bd.T)
    q, k, v, xr, g2 = _tc_proj(x, wcat_t, bcat, wem16)

    z1d = jnp.zeros((_DEN_PAD,), jnp.float32)
    z2d = jnp.zeros((_SLAB_ROWS, _D), jnp.float32)
    exb, den = _sc_pass1(q, k, src, dst, attr, g2, z1d)
    out_msg, cv = _sc_pass2(v, src, dst, attr, exb, den, z1d, z2d)

    cvn = cv.reshape(_NC, 2, _CV_PAD // 2)[:, :, :_RN * _H]
    cvn = cvn.reshape(_N, _H)
    w_out = (Wbeta[0, :_D] + Wbeta[0, 2 * _D:]).reshape(_D, 1)
    w_xr = (Wbeta[0, _D:2 * _D] - Wbeta[0, 2 * _D:]).reshape(_D, 1)
    return _tc_finish(out_msg, cvn, xr, we_bd, w_out, w_xr, Wt.T,
                      bt.reshape(1, _F), gamma.reshape(1, _F),
                      beta_bn.reshape(1, _F))
